# baseline (speedup 1.0000x reference)
import functools

import numpy as np

import jax
import jax.numpy as jnp
from jax import lax
from jax.experimental import pallas as pl
from jax.experimental.pallas import tpu as pltpu
from jax.experimental.pallas import tpu_sc as plsc

B = 1024
S = 50
S1 = 14
S2 = S - S1
E = 128
H = 256
B2 = 2 * B
FEAT = 2 * E + 4 * H
NW = 32
CH = 64
GRP = 5
BF = jnp.bfloat16
F32 = jnp.float32


def _make_gather_body(steps):
    rpw = B * steps // NW
    nchunk = rpw // CH

    def body(vac_tab, res_tab, vac_idx, res_idx, out, idx_v, rows_v,
             sem, sem_out):
        cid = lax.axis_index("c")
        sid = lax.axis_index("s")
        wid = sid * 2 + cid
        base = wid * rpw

        def run(tab, idx_hbm, boff):
            pltpu.sync_copy(idx_hbm.at[wid], idx_v)
            for g0 in range(0, nchunk, GRP):
                grp = min(GRP, nchunk - g0)
                gathers = [
                    pltpu.async_copy(tab.at[idx_v.at[g0 + j]],
                                     rows_v.at[pl.ds(j * CH, CH)], sem)
                    for j in range(grp)
                ]
                for cp in gathers:
                    cp.wait()
                scatters = []
                for j in range(grp):
                    r0 = base + (g0 + j) * CH
                    comb = r0 + (r0 // B) * B + boff
                    scatters.append(
                        pltpu.async_copy(rows_v.at[pl.ds(j * CH, CH)],
                                         out.at[pl.ds(comb, CH)], sem_out))
                for cp in scatters:
                    cp.wait()

        run(vac_tab, vac_idx, 0)
        run(res_tab, res_idx, B)

    return body, nchunk


@functools.cache
def _sc_gather(steps):
    body, nchunk = _make_gather_body(steps)
    return pl.kernel(
        body,
        out_type=jax.ShapeDtypeStruct((steps * B2, E), F32),
        mesh=plsc.VectorSubcoreMesh(core_axis_name="c", subcore_axis_name="s"),
        scratch_types=[
            pltpu.VMEM((nchunk, CH), jnp.int32),
            pltpu.VMEM((GRP * CH, E), F32),
            pltpu.SemaphoreType.DMA,
            pltpu.SemaphoreType.DMA,
        ],
    )


def _cell(x, h, c, zb_s, wz_ref, b_ref):
    zb_s[:, 0:E] = x.astype(BF)
    zb_s[:, E:E + H] = h
    g0 = (jnp.dot(zb_s[...], wz_ref[...],
                  preferred_element_type=F32).astype(BF)
          + b_ref[...])
    half = BF(0.5)
    u = jnp.tanh(g0[:, 0:3 * H])
    gi = u[:, 0:H] * half + half
    gf = u[:, H:2 * H] * half + half
    go = u[:, 2 * H:3 * H] * half + half
    gg = jnp.tanh(g0[:, 3 * H:4 * H])
    cn = gf * c + gi * gg
    hn = go * jnp.tanh(cn)
    return hn, cn


def _steps_block(unroll, x_ref, wz_ref, b_ref, zb_s, h_s, c_s, rmax_s, rsum_s,
                 emax_s, esum_s):
    h = h_s[...]
    c = c_s[...]
    xs = [x_ref[u] for u in range(unroll)]
    hs = []
    for u in range(unroll):
        h, c = _cell(xs[u], h, c, zb_s, wz_ref, b_ref)
        hs.append(h)
    h_s[...] = h
    c_s[...] = c
    hmax = hs[0]
    hsum = hs[0]
    for u in range(1, unroll):
        hmax = jnp.maximum(hmax, hs[u])
        hsum = hsum + hs[u]
    xmax = xs[0]
    xsum = xs[0]
    for u in range(1, unroll):
        xmax = jnp.maximum(xmax, xs[u])
        xsum = xsum + xs[u]
    rmax_s[...] = jnp.maximum(rmax_s[...], hmax)
    rsum_s[...] = rsum_s[...] + hsum.astype(F32)
    emax_s[...] = jnp.maximum(emax_s[...], xmax)
    esum_s[...] = esum_s[...] + xsum


def _lstm1_body(unroll, x_ref, wz_ref, b_ref,
                h_o, c_o, rmax_o, rsum_o, emax_o, esum_o, zb_s):
    t = pl.program_id(0)

    @pl.when(t == 0)
    def _():
        h_o[...] = jnp.zeros((B2, H), BF)
        c_o[...] = jnp.zeros((B2, H), BF)
        rmax_o[...] = jnp.full((B2, H), -jnp.inf, BF)
        rsum_o[...] = jnp.zeros((B2, H), F32)
        emax_o[...] = jnp.full((B2, E), -jnp.inf, F32)
        esum_o[...] = jnp.zeros((B2, E), F32)

    _steps_block(unroll, x_ref, wz_ref, b_ref, zb_s, h_o, c_o, rmax_o, rsum_o,
                 emax_o, esum_o)


def _lstm2_body(unroll, nit, x_ref, wz_ref, b_ref, w1_ref, b1_ref, w2_ref,
                b2_ref, h_i, c_i, rmax_i, rsum_i, emax_i, esum_i, out_ref,
                h_s, c_s, rmax_s, rsum_s, emax_s, esum_s, cat_s, zb_s):
    t = pl.program_id(0)

    @pl.when(t == 0)
    def _():
        h_s[...] = h_i[...]
        c_s[...] = c_i[...]
        rmax_s[...] = rmax_i[...]
        rsum_s[...] = rsum_i[...]
        emax_s[...] = emax_i[...]
        esum_s[...] = esum_i[...]

    _steps_block(unroll, x_ref, wz_ref, b_ref, zb_s, h_s, c_s, rmax_s, rsum_s,
                 emax_s, esum_s)

    @pl.when(t == nit - 1)
    def _():
        inv = F32(1.0 / B)
        emax = emax_s[...]
        esum = esum_s[...] * inv
        rmax = rmax_s[...].astype(F32)
        rsum = rsum_s[...] * inv
        hT = h_s[...].astype(F32)
        cT = c_s[...].astype(F32)
        for half in range(2):
            off = half * FEAT
            r0, r1 = half * B, (half + 1) * B
            cat_s[:, off + 0:off + E] = emax[r0:r1, :]
            cat_s[:, off + E:off + 2 * E] = esum[r0:r1, :]
            cat_s[:, off + 2 * E:off + 2 * E + H] = rmax[r0:r1, :]
            cat_s[:, off + 2 * E + H:off + 2 * E + 2 * H] = rsum[r0:r1, :]
            cat_s[:, off + 2 * E + 2 * H:off + 2 * E + 3 * H] = hT[r0:r1, :]
            cat_s[:, off + 2 * E + 3 * H:off + 2 * E + 4 * H] = cT[r0:r1, :]
        cat = cat_s[...].astype(BF)
        h1 = jnp.maximum(
            jnp.dot(cat, w1_ref[...], preferred_element_type=F32)
            + b1_ref[...], 0.0)
        out_ref[...] = jax.nn.sigmoid(
            jnp.dot(h1.astype(BF), w2_ref[...], preferred_element_type=F32)
            + b2_ref[...])


def _full(shape, dtype):
    return pl.BlockSpec(shape, lambda t: tuple(0 for _ in shape))


_CARRY_SHAPES = [
    ((B2, H), BF), ((B2, H), BF), ((B2, H), BF),
    ((B2, H), F32), ((B2, E), F32), ((B2, E), F32),
]

U1 = 7
U2 = 6

_lstm1 = pl.pallas_call(
    functools.partial(_lstm1_body, U1),
    grid=(S1 // U1,),
    in_specs=[
        pl.BlockSpec((U1, B2, E), lambda t: (t, 0, 0)),
        _full((E + H, 4 * H), BF),
        _full((1, 4 * H), BF),
    ],
    out_specs=[_full(shp, dt) for shp, dt in _CARRY_SHAPES],
    out_shape=[jax.ShapeDtypeStruct(shp, dt) for shp, dt in _CARRY_SHAPES],
    scratch_shapes=[
        pltpu.VMEM((B2, E + H), BF),
    ],
    compiler_params=pltpu.CompilerParams(dimension_semantics=("arbitrary",)),
)

_lstm2 = pl.pallas_call(
    functools.partial(_lstm2_body, U2, S2 // U2),
    grid=(S2 // U2,),
    in_specs=[
        pl.BlockSpec((U2, B2, E), lambda t: (t, 0, 0)),
        _full((E + H, 4 * H), BF),
        _full((1, 4 * H), BF),
        _full((2 * FEAT, 512), BF),
        _full((1, 512), F32),
        _full((512, 128), BF),
        _full((1, 128), F32),
    ] + [_full(shp, dt) for shp, dt in _CARRY_SHAPES],
    out_specs=pl.BlockSpec((B, 128), lambda t: (0, 0)),
    out_shape=jax.ShapeDtypeStruct((B, 128), F32),
    scratch_shapes=[
        pltpu.VMEM((B2, H), BF),
        pltpu.VMEM((B2, H), BF),
        pltpu.VMEM((B2, H), BF),
        pltpu.VMEM((B2, H), F32),
        pltpu.VMEM((B2, E), F32),
        pltpu.VMEM((B2, E), F32),
        pltpu.VMEM((B, 2 * FEAT), F32),
        pltpu.VMEM((B2, E + H), BF),
    ],
    compiler_params=pltpu.CompilerParams(dimension_semantics=("arbitrary",)),
)

_GATE_PERM = np.concatenate([
    np.arange(0, 2 * H), np.arange(3 * H, 4 * H), np.arange(2 * H, 3 * H)])


def kernel(vac_text, res_text, vac_table, res_table, W_ih, W_hh, b_ih, b_hh,
           fc1_W, fc1_b, fc2_W, fc2_b):
    vt = vac_text.astype(jnp.int32).T
    rt = res_text.astype(jnp.int32).T
    n1 = B * S1 // NW // CH
    n2 = B * S2 // NW // CH
    x1 = _sc_gather(S1)(vac_table, res_table,
                        vt[:S1].reshape(NW, n1, CH),
                        rt[:S1].reshape(NW, n1, CH)).reshape(S1, B2, E)
    x2 = _sc_gather(S2)(vac_table, res_table,
                        vt[S1:].reshape(NW, n2, CH),
                        rt[S1:].reshape(NW, n2, CH)).reshape(S2, B2, E)
    b = (b_ih + b_hh)[_GATE_PERM]
    bias = (jnp.concatenate([b[:3 * H] * 0.5, b[3 * H:]])
            .reshape(1, 4 * H).astype(BF))
    wz = jnp.concatenate([W_ih.T, W_hh.T], axis=0)[:, _GATE_PERM]
    wz = jnp.concatenate([wz[:, :3 * H] * 0.5, wz[:, 3 * H:]],
                         axis=1).astype(BF)
    carry = _lstm1(x1, wz, bias)
    return _lstm2(x2, wz, bias,
                  fc1_W.T.astype(BF), fc1_b.reshape(1, -1),
                  fc2_W.T.astype(BF), fc2_b.reshape(1, -1), *carry)


# speedup vs baseline: 1.1226x; 1.1226x over previous
import functools

import numpy as np

import jax
import jax.numpy as jnp
from jax import lax
from jax.experimental import pallas as pl
from jax.experimental.pallas import tpu as pltpu
from jax.experimental.pallas import tpu_sc as plsc

B = 1024
S = 50
S1 = 20
S2 = S - S1
E = 128
H = 256
B2 = 2 * B
FEAT = 2 * E + 4 * H
NW = 32
CH = 64
GRP = 5
BF = jnp.bfloat16
F32 = jnp.float32


def _make_gather_body(steps):
    rpw = B * steps // NW
    nchunk = rpw // CH

    def body(vac_tab, res_tab, vac_idx, res_idx, out, idx_v, rows_v,
             sem, sem_out):
        cid = lax.axis_index("c")
        sid = lax.axis_index("s")
        wid = sid * 2 + cid
        base = wid * rpw

        def run(tab, idx_hbm, boff):
            pltpu.sync_copy(idx_hbm.at[wid], idx_v)
            for g0 in range(0, nchunk, GRP):
                grp = min(GRP, nchunk - g0)
                gathers = [
                    pltpu.async_copy(tab.at[idx_v.at[g0 + j]],
                                     rows_v.at[pl.ds(j * CH, CH)], sem)
                    for j in range(grp)
                ]
                for cp in gathers:
                    cp.wait()
                scatters = []
                for j in range(grp):
                    r0 = base + (g0 + j) * CH
                    comb = r0 + (r0 // B) * B + boff
                    scatters.append(
                        pltpu.async_copy(rows_v.at[pl.ds(j * CH, CH)],
                                         out.at[pl.ds(comb, CH)], sem_out))
                for cp in scatters:
                    cp.wait()

        run(vac_tab, vac_idx, 0)
        run(res_tab, res_idx, B)

    return body, nchunk


@functools.cache
def _sc_gather(steps):
    body, nchunk = _make_gather_body(steps)
    return pl.kernel(
        body,
        out_type=jax.ShapeDtypeStruct((steps * B2, E), F32),
        mesh=plsc.VectorSubcoreMesh(core_axis_name="c", subcore_axis_name="s"),
        scratch_types=[
            pltpu.VMEM((nchunk, CH), jnp.int32),
            pltpu.VMEM((GRP * CH, E), F32),
            pltpu.SemaphoreType.DMA,
            pltpu.SemaphoreType.DMA,
        ],
    )


def _cell(x, h, c, zb_s, wz_ref, b_ref):
    zb_s[:, 0:E] = x.astype(BF)
    zb_s[:, E:E + H] = h
    g0 = (jnp.dot(zb_s[...], wz_ref[...],
                  preferred_element_type=F32).astype(BF)
          + b_ref[...])
    half = BF(0.5)
    u = jnp.tanh(g0[:, 0:3 * H])
    gi = u[:, 0:H] * half + half
    gf = u[:, H:2 * H] * half + half
    go = u[:, 2 * H:3 * H] * half + half
    gg = jnp.tanh(g0[:, 3 * H:4 * H])
    cn = gf * c + gi * gg
    hn = go * jnp.tanh(cn)
    return hn, cn


def _steps_block(unroll, x_ref, wz_ref, b_ref, zb_s, h_s, c_s, rmax_s, rsum_s,
                 emax_s, esum_s):
    h = h_s[...]
    c = c_s[...]
    xs = [x_ref[u] for u in range(unroll)]
    hs = []
    for u in range(unroll):
        h, c = _cell(xs[u], h, c, zb_s, wz_ref, b_ref)
        hs.append(h)
    h_s[...] = h
    c_s[...] = c
    hmax = hs[0]
    hsum = hs[0]
    for u in range(1, unroll):
        hmax = jnp.maximum(hmax, hs[u])
        hsum = hsum + hs[u]
    xmax = xs[0]
    xsum = xs[0]
    for u in range(1, unroll):
        xmax = jnp.maximum(xmax, xs[u])
        xsum = xsum + xs[u]
    rmax_s[...] = jnp.maximum(rmax_s[...], hmax)
    rsum_s[...] = rsum_s[...] + hsum.astype(F32)
    emax_s[...] = jnp.maximum(emax_s[...], xmax)
    esum_s[...] = esum_s[...] + xsum


def _lstm1_body(unroll, x_ref, wz_ref, b_ref,
                h_o, c_o, rmax_o, rsum_o, emax_o, esum_o, zb_s):
    t = pl.program_id(0)

    @pl.when(t == 0)
    def _():
        h_o[...] = jnp.zeros((B2, H), BF)
        c_o[...] = jnp.zeros((B2, H), BF)
        rmax_o[...] = jnp.full((B2, H), -jnp.inf, BF)
        rsum_o[...] = jnp.zeros((B2, H), F32)
        emax_o[...] = jnp.full((B2, E), -jnp.inf, F32)
        esum_o[...] = jnp.zeros((B2, E), F32)

    _steps_block(unroll, x_ref, wz_ref, b_ref, zb_s, h_o, c_o, rmax_o, rsum_o,
                 emax_o, esum_o)


def _lstm2_body(unroll, nit, x_ref, wz_ref, b_ref, w1_ref, b1_ref, w2_ref,
                b2_ref, h_i, c_i, rmax_i, rsum_i, emax_i, esum_i, out_ref,
                h_s, c_s, rmax_s, rsum_s, emax_s, esum_s, cat_s, zb_s):
    t = pl.program_id(0)

    @pl.when(t == 0)
    def _():
        h_s[...] = h_i[...]
        c_s[...] = c_i[...]
        rmax_s[...] = rmax_i[...]
        rsum_s[...] = rsum_i[...]
        emax_s[...] = emax_i[...]
        esum_s[...] = esum_i[...]

    _steps_block(unroll, x_ref, wz_ref, b_ref, zb_s, h_s, c_s, rmax_s, rsum_s,
                 emax_s, esum_s)

    @pl.when(t == nit - 1)
    def _():
        inv = F32(1.0 / B)
        emax = emax_s[...]
        esum = esum_s[...] * inv
        rmax = rmax_s[...].astype(F32)
        rsum = rsum_s[...] * inv
        hT = h_s[...].astype(F32)
        cT = c_s[...].astype(F32)
        for half in range(2):
            off = half * FEAT
            r0, r1 = half * B, (half + 1) * B
            cat_s[:, off + 0:off + E] = emax[r0:r1, :]
            cat_s[:, off + E:off + 2 * E] = esum[r0:r1, :]
            cat_s[:, off + 2 * E:off + 2 * E + H] = rmax[r0:r1, :]
            cat_s[:, off + 2 * E + H:off + 2 * E + 2 * H] = rsum[r0:r1, :]
            cat_s[:, off + 2 * E + 2 * H:off + 2 * E + 3 * H] = hT[r0:r1, :]
            cat_s[:, off + 2 * E + 3 * H:off + 2 * E + 4 * H] = cT[r0:r1, :]
        cat = cat_s[...].astype(BF)
        h1 = jnp.maximum(
            jnp.dot(cat, w1_ref[...], preferred_element_type=F32)
            + b1_ref[...], 0.0)
        out_ref[...] = jax.nn.sigmoid(
            jnp.dot(h1.astype(BF), w2_ref[...], preferred_element_type=F32)
            + b2_ref[...])


def _full(shape, dtype):
    return pl.BlockSpec(shape, lambda t: tuple(0 for _ in shape))


_CARRY_SHAPES = [
    ((B2, H), BF), ((B2, H), BF), ((B2, H), BF),
    ((B2, H), F32), ((B2, E), F32), ((B2, E), F32),
]

U1 = 5
U2 = 5

_lstm1 = pl.pallas_call(
    functools.partial(_lstm1_body, U1),
    grid=(S1 // U1,),
    in_specs=[
        pl.BlockSpec((U1, B2, E), lambda t: (t, 0, 0)),
        _full((E + H, 4 * H), BF),
        _full((1, 4 * H), BF),
    ],
    out_specs=[_full(shp, dt) for shp, dt in _CARRY_SHAPES],
    out_shape=[jax.ShapeDtypeStruct(shp, dt) for shp, dt in _CARRY_SHAPES],
    scratch_shapes=[
        pltpu.VMEM((B2, E + H), BF),
    ],
    compiler_params=pltpu.CompilerParams(dimension_semantics=("arbitrary",)),
)

_lstm2 = pl.pallas_call(
    functools.partial(_lstm2_body, U2, S2 // U2),
    grid=(S2 // U2,),
    in_specs=[
        pl.BlockSpec((U2, B2, E), lambda t: (t, 0, 0)),
        _full((E + H, 4 * H), BF),
        _full((1, 4 * H), BF),
        _full((2 * FEAT, 512), BF),
        _full((1, 512), F32),
        _full((512, 128), BF),
        _full((1, 128), F32),
    ] + [_full(shp, dt) for shp, dt in _CARRY_SHAPES],
    out_specs=pl.BlockSpec((B, 128), lambda t: (0, 0)),
    out_shape=jax.ShapeDtypeStruct((B, 128), F32),
    scratch_shapes=[
        pltpu.VMEM((B2, H), BF),
        pltpu.VMEM((B2, H), BF),
        pltpu.VMEM((B2, H), BF),
        pltpu.VMEM((B2, H), F32),
        pltpu.VMEM((B2, E), F32),
        pltpu.VMEM((B2, E), F32),
        pltpu.VMEM((B, 2 * FEAT), F32),
        pltpu.VMEM((B2, E + H), BF),
    ],
    compiler_params=pltpu.CompilerParams(dimension_semantics=("arbitrary",)),
)

_GATE_PERM = np.concatenate([
    np.arange(0, 2 * H), np.arange(3 * H, 4 * H), np.arange(2 * H, 3 * H)])


def kernel(vac_text, res_text, vac_table, res_table, W_ih, W_hh, b_ih, b_hh,
           fc1_W, fc1_b, fc2_W, fc2_b):
    vt = vac_text.astype(jnp.int32).T
    rt = res_text.astype(jnp.int32).T
    n1 = B * S1 // NW // CH
    n2 = B * S2 // NW // CH
    x1 = _sc_gather(S1)(vac_table, res_table,
                        vt[:S1].reshape(NW, n1, CH),
                        rt[:S1].reshape(NW, n1, CH)).reshape(S1, B2, E)
    x2 = _sc_gather(S2)(vac_table, res_table,
                        vt[S1:].reshape(NW, n2, CH),
                        rt[S1:].reshape(NW, n2, CH)).reshape(S2, B2, E)
    b = (b_ih + b_hh)[_GATE_PERM]
    bias = (jnp.concatenate([b[:3 * H] * 0.5, b[3 * H:]])
            .reshape(1, 4 * H).astype(BF))
    wz = jnp.concatenate([W_ih.T, W_hh.T], axis=0)[:, _GATE_PERM]
    wz = jnp.concatenate([wz[:, :3 * H] * 0.5, wz[:, 3 * H:]],
                         axis=1).astype(BF)
    carry = _lstm1(x1, wz, bias)
    return _lstm2(x2, wz, bias,
                  fc1_W.T.astype(BF), fc1_b.reshape(1, -1),
                  fc2_W.T.astype(BF), fc2_b.reshape(1, -1), *carry)
